# baseline (speedup 1.0000x reference)
import functools

import jax
import jax.numpy as jnp
import numpy as np
from jax import lax
from jax.experimental import pallas as pl
from jax.experimental.pallas import tpu as pltpu
from jax.experimental.pallas import tpu_sc as plsc


def _pos_encoding_np(seq_len: int, d_model: int) -> np.ndarray:
    pos = np.arange(seq_len)[:, np.newaxis]
    i = np.arange(d_model)[np.newaxis, :]
    angle_rates = 1 / np.power(10000, 2 * (i // 2) / np.float32(d_model))
    angle_rads = pos * angle_rates
    angle_rads[:, 0::2] = np.sin(angle_rads[:, 0::2])
    angle_rads[:, 1::2] = np.cos(angle_rads[:, 1::2])
    return angle_rads.astype(np.float32)


@jax.jit
def _embed(x, token_table, pos_tiled):
    B, S = x.shape
    V, D = token_table.shape

    info = plsc.get_sparse_core_info()
    NC, NS = info.num_cores, info.num_subcores
    NW = NC * NS

    rows_total = B * S
    rows_per_w = rows_total // NW
    assert rows_per_w * NW == rows_total
    G = S // 2
    SEQ_PER_CHUNK = 8
    CH = SEQ_PER_CHUNK * S
    GPC = CH // G
    n_chunks = rows_per_w // CH
    assert n_chunks * CH == rows_per_w
    assert pos_tiled.shape == (CH, D)

    idx = x.reshape(NW, rows_per_w // G, G)

    mesh = plsc.VectorSubcoreMesh(core_axis_name="c", subcore_axis_name="s")

    @functools.partial(
        pl.kernel,
        mesh=mesh,
        out_type=jax.ShapeDtypeStruct((rows_total, D), jnp.float32),
        scratch_types=[
            pltpu.VMEM((rows_per_w // G, G), jnp.int32),
            pltpu.VMEM((CH, D), jnp.float32),
            pltpu.SemaphoreType.DMA,
        ],
        compiler_params=pltpu.CompilerParams(use_tc_tiling_on_sc=False),
    )
    def k(table_hbm, idx_hbm, pos_hbm, out_hbm, idx_v, buf, sem):
        sid = lax.axis_index("s")
        wid = lax.axis_index("c") * NS + sid
        base = wid * rows_per_w

        pltpu.sync_copy(idx_hbm.at[wid], idx_v)

        def chunk(c, _):
            pltpu.sync_copy(pos_hbm, buf)
            ds = []
            for j in range(GPC):
                ds.append(pltpu.async_copy(
                    table_hbm.at[idx_v.at[c * GPC + j]],
                    buf.at[pl.ds(j * G, G)], sem, add=True))
            for d in ds:
                d.wait()
            pltpu.sync_copy(buf, out_hbm.at[pl.ds(base + c * CH, CH)])
            return 0

        lax.fori_loop(0, n_chunks, chunk, 0)

    return k(token_table, idx, pos_tiled)


def kernel(x, token_table):
    B, S = x.shape
    D = token_table.shape[1]
    pos_enc = _pos_encoding_np(S, D)
    pos_tiled = jnp.asarray(np.tile(pos_enc, (8, 1)))
    out = _embed(x, token_table, pos_tiled)
    return out.reshape(B, S, D)


# speedup vs baseline: 1.0387x; 1.0387x over previous
import functools

import jax
import jax.numpy as jnp
import numpy as np
from jax import lax
from jax.experimental import pallas as pl
from jax.experimental.pallas import tpu as pltpu
from jax.experimental.pallas import tpu_sc as plsc


def _pos_encoding_np(seq_len: int, d_model: int) -> np.ndarray:
    pos = np.arange(seq_len)[:, np.newaxis]
    i = np.arange(d_model)[np.newaxis, :]
    angle_rates = 1 / np.power(10000, 2 * (i // 2) / np.float32(d_model))
    angle_rads = pos * angle_rates
    angle_rads[:, 0::2] = np.sin(angle_rads[:, 0::2])
    angle_rads[:, 1::2] = np.cos(angle_rads[:, 1::2])
    return angle_rads.astype(np.float32)


@jax.jit
def _embed(x, token_table, pos_tiled):
    B, S = x.shape
    V, D = token_table.shape

    info = plsc.get_sparse_core_info()
    NC, NS = info.num_cores, info.num_subcores
    NW = NC * NS

    rows_total = B * S
    rows_per_w = rows_total // NW
    assert rows_per_w * NW == rows_total
    G = S // 2
    SEQ_PER_CHUNK = 8
    CH = SEQ_PER_CHUNK * S
    GPC = CH // G
    n_chunks = rows_per_w // CH
    assert n_chunks * CH == rows_per_w
    assert pos_tiled.shape == (CH, D)

    idx = x.reshape(NW, rows_per_w // G, G)

    mesh = plsc.VectorSubcoreMesh(core_axis_name="c", subcore_axis_name="s")

    @functools.partial(
        pl.kernel,
        mesh=mesh,
        out_type=jax.ShapeDtypeStruct((rows_total, D), jnp.float32),
        scratch_types=[
            pltpu.VMEM((rows_per_w // G, G), jnp.int32),
            pltpu.VMEM_SHARED((CH, D), jnp.float32),
            pltpu.VMEM((CH, D), jnp.float32),
            pltpu.SemaphoreType.DMA,
        ],
        compiler_params=pltpu.CompilerParams(use_tc_tiling_on_sc=False),
    )
    def k(table_hbm, idx_hbm, pos_hbm, out_hbm, idx_v, pos_sh, buf, sem):
        sid = lax.axis_index("s")
        wid = lax.axis_index("c") * NS + sid
        base = wid * rows_per_w

        @pl.when(sid == 0)
        def _():
            pltpu.sync_copy(pos_hbm, pos_sh)
        pltpu.sync_copy(idx_hbm.at[wid], idx_v)
        plsc.subcore_barrier()

        def chunk(c, _):
            pltpu.sync_copy(pos_sh, buf)
            ds = []
            for j in range(GPC):
                ds.append(pltpu.async_copy(
                    table_hbm.at[idx_v.at[c * GPC + j]],
                    buf.at[pl.ds(j * G, G)], sem, add=True))
            for d in ds:
                d.wait()
            pltpu.sync_copy(buf, out_hbm.at[pl.ds(base + c * CH, CH)])
            return 0

        lax.fori_loop(0, n_chunks, chunk, 0)

    return k(token_table, idx, pos_tiled)


def kernel(x, token_table):
    B, S = x.shape
    D = token_table.shape[1]
    pos_enc = _pos_encoding_np(S, D)
    pos_tiled = jnp.asarray(np.tile(pos_enc, (8, 1)))
    out = _embed(x, token_table, pos_tiled)
    return out.reshape(B, S, D)
